# baseline (speedup 1.0000x reference)
import functools

import jax
import jax.numpy as jnp
from jax.experimental import pallas as pl
from jax.experimental.pallas import tpu as pltpu

_TARGET_BLOCK_BYTES = 3 * 1024 * 1024


def _charb_step(x_ref, y_ref, out_ref, acc_ref, *, eps, block_rows,
                steps_per_split):
    t = pl.program_id(1)
    w = x_ref.shape[-1]
    d = x_ref[...] - y_ref[...]
    v = d * d + jnp.float32(eps)
    err = v * jax.lax.rsqrt(v)
    folded = err.reshape(block_rows // 8, 8, w).sum(axis=0)

    @pl.when(t == 0)
    def _():
        acc_ref[...] = folded

    @pl.when(t > 0)
    def _():
        acc_ref[...] += folded

    @pl.when(t == steps_per_split - 1)
    def _():
        out_ref[0] = acc_ref[...]


def _pick_block(total_rows, row_bytes):
    best = 8
    b = 8
    while b <= total_rows:
        if total_rows % b == 0 and b * row_bytes <= _TARGET_BLOCK_BYTES:
            best = b
        b += 8
    return best


def kernel(outputs, targets):
    eps = 1e-6
    shape = outputs.shape
    w = shape[-1]
    rows = 1
    for s in shape[:-1]:
        rows *= s
    n = rows * w

    x = outputs.reshape(rows, w)
    y = targets.reshape(rows, w)

    row_bytes = w * jnp.dtype(x.dtype).itemsize
    nsplit = 2 if rows % 16 == 0 else 1
    split_rows = rows // nsplit
    block_rows = _pick_block(split_rows, row_bytes)
    steps_per_split = split_rows // block_rows

    body = functools.partial(_charb_step, eps=eps, block_rows=block_rows,
                             steps_per_split=steps_per_split)

    partials = pl.pallas_call(
        body,
        out_shape=jax.ShapeDtypeStruct((nsplit, 8, w), jnp.float32),
        grid=(nsplit, steps_per_split),
        in_specs=[
            pl.BlockSpec((block_rows, w),
                         lambda s, t: (s * steps_per_split + t, 0)),
            pl.BlockSpec((block_rows, w),
                         lambda s, t: (s * steps_per_split + t, 0)),
        ],
        out_specs=pl.BlockSpec((1, 8, w), lambda s, t: (s, 0, 0)),
        scratch_shapes=[pltpu.VMEM((8, w), jnp.float32)],
        compiler_params=pltpu.CompilerParams(
            dimension_semantics=("parallel", "arbitrary"),
        ),
    )(x, y)

    return jnp.sum(partials) / jnp.float32(n)


# speedup vs baseline: 1.1724x; 1.1724x over previous
import functools

import jax
import jax.numpy as jnp
from jax.experimental import pallas as pl
from jax.experimental.pallas import tpu as pltpu

_TARGET_BLOCK_BYTES = 2 * 1024 * 1024


def _charb_step(x_ref, y_ref, out_ref, acc_ref, *, eps, block_rows,
                num_tiles, inv_n):
    t = pl.program_id(0)
    w = x_ref.shape[-1]
    d = x_ref[...] - y_ref[...]
    v = d * d + jnp.float32(eps)
    err = v * jax.lax.rsqrt(v)
    folded = err.reshape(block_rows // 8, 8, w).sum(axis=0)

    @pl.when(t == 0)
    def _():
        acc_ref[...] = folded

    @pl.when(t > 0)
    def _():
        acc_ref[...] += folded

    @pl.when(t == num_tiles - 1)
    def _():
        out_ref[0, 0] = jnp.sum(acc_ref[...]) * jnp.float32(inv_n)


def _pick_block(total_rows, row_bytes):
    best = 8
    b = 8
    while b <= total_rows:
        if total_rows % b == 0 and b * row_bytes <= _TARGET_BLOCK_BYTES:
            best = b
        b += 8
    return best


def kernel(outputs, targets):
    eps = 1e-6
    shape = outputs.shape
    w = shape[-1]
    rows = 1
    for s in shape[:-1]:
        rows *= s
    n = rows * w

    x = outputs.reshape(rows, w)
    y = targets.reshape(rows, w)

    row_bytes = w * jnp.dtype(x.dtype).itemsize
    block_rows = _pick_block(rows, row_bytes)
    num_tiles = rows // block_rows

    body = functools.partial(_charb_step, eps=eps, block_rows=block_rows,
                             num_tiles=num_tiles, inv_n=1.0 / n)

    loss = pl.pallas_call(
        body,
        out_shape=jax.ShapeDtypeStruct((1, 1), jnp.float32),
        grid=(num_tiles,),
        in_specs=[
            pl.BlockSpec((block_rows, w), lambda t: (t, 0)),
            pl.BlockSpec((block_rows, w), lambda t: (t, 0)),
        ],
        out_specs=pl.BlockSpec(memory_space=pltpu.SMEM),
        scratch_shapes=[pltpu.VMEM((8, w), jnp.float32)],
        compiler_params=pltpu.CompilerParams(
            dimension_semantics=("arbitrary",),
        ),
    )(x, y)

    return loss.reshape(())
